# baseline (speedup 1.0000x reference)
import functools

import jax
import jax.numpy as jnp
from jax import lax
from jax.experimental import pallas as pl
from jax.experimental.pallas import tpu as pltpu
from jax.experimental.pallas import tpu_sc as plsc

N = 10000
E = 320000
D_IN = 128
D_HID = 16
D_OUT = 128

NC = 2
NS = 16
NW = NC * NS
CHUNK = 64
JCH = 159
EW = JCH * CHUNK
EPAD = EW * NW
NPAD = 10240
RPT = NPAD // NS

_MESH = plsc.VectorSubcoreMesh(core_axis_name="c", subcore_axis_name="s")


_SC_PARAMS = pltpu.CompilerParams(use_tc_tiling_on_sc=False)


@functools.partial(
    pl.kernel,
    out_type=jax.ShapeDtypeStruct((NC, NPAD, D_HID), jnp.float32),
    mesh=_MESH,
    compiler_params=_SC_PARAMS,
    scratch_types=[
        pltpu.VMEM((JCH, CHUNK), jnp.int32),
        pltpu.VMEM((CHUNK, D_HID), jnp.float32),
        pltpu.VMEM_SHARED((NPAD, D_HID), jnp.float32),
        pltpu.SemaphoreType.DMA,
    ],
)
def _sc_degree(dst_hbm, ones_hbm, zeros_hbm, out_hbm, didx, ones_v, acc, sem):
    cid = lax.axis_index("c")
    sid = lax.axis_index("s")
    wid = sid * NC + cid
    pltpu.sync_copy(zeros_hbm, acc.at[pl.ds(sid * RPT, RPT)])
    pltpu.sync_copy(dst_hbm.at[wid], didx)
    pltpu.sync_copy(ones_hbm, ones_v)
    plsc.subcore_barrier()

    @pl.loop(0, JCH)
    def _(j):
        pltpu.async_copy(ones_v, acc.at[didx.at[j]], sem, add=True)

    @pl.loop(0, JCH)
    def _(j):
        pltpu.make_async_copy(ones_v, acc.at[didx.at[j]], sem).wait()

    plsc.subcore_barrier()
    pltpu.sync_copy(acc.at[pl.ds(sid * RPT, RPT)],
                    out_hbm.at[cid, pl.ds(sid * RPT, RPT)])


@functools.partial(
    pl.kernel,
    out_type=jax.ShapeDtypeStruct((NC, NPAD, D_HID), jnp.float32),
    mesh=_MESH,
    compiler_params=_SC_PARAMS,
    scratch_types=[
        pltpu.VMEM((JCH, CHUNK), jnp.int32),
        pltpu.VMEM((JCH, CHUNK), jnp.int32),
        pltpu.VMEM((4, CHUNK, D_HID), jnp.float32),
        pltpu.VMEM_SHARED((NPAD, D_HID), jnp.float32),
        [pltpu.SemaphoreType.DMA] * 4,
    ],
)
def _sc_spmm(g_hbm, src_hbm, dst_hbm, zeros_hbm, out_hbm,
             sidx, didx, rows, acc, gsem):
    cid = lax.axis_index("c")
    sid = lax.axis_index("s")
    wid = sid * NC + cid
    pltpu.sync_copy(zeros_hbm, acc.at[pl.ds(sid * RPT, RPT)])
    pltpu.sync_copy(src_hbm.at[wid], sidx)
    pltpu.sync_copy(dst_hbm.at[wid], didx)
    plsc.subcore_barrier()

    def _gather(j, b):
        pltpu.async_copy(g_hbm.at[sidx.at[j]], rows.at[b], gsem[b])

    def _wait_scatter(j, b):
        pltpu.make_async_copy(g_hbm.at[sidx.at[j]], rows.at[b], gsem[b]).wait()
        pltpu.sync_copy(rows.at[b], acc.at[didx.at[j]], add=True)

    _gather(0, 0)
    _gather(1, 1)
    _gather(2, 2)

    @pl.loop(0, JCH - 6, step=4)
    def _(j):
        _gather(j + 3, 3)
        _wait_scatter(j, 0)
        _gather(j + 4, 0)
        _wait_scatter(j + 1, 1)
        _gather(j + 5, 1)
        _wait_scatter(j + 2, 2)
        _gather(j + 6, 2)
        _wait_scatter(j + 3, 3)

    _wait_scatter(JCH - 3, 0)
    _wait_scatter(JCH - 2, 1)
    _wait_scatter(JCH - 1, 2)

    plsc.subcore_barrier()
    pltpu.sync_copy(acc.at[pl.ds(sid * RPT, RPT)],
                    out_hbm.at[cid, pl.ds(sid * RPT, RPT)])


RB = 2000
_F32 = jnp.float32


def _pre_body(x_ref, deg_ref, w1_ref, t1w_ref, t1b_ref,
              dis_ref, g1_ref, id1_ref):
    x = x_ref[...]
    dis = lax.rsqrt(deg_ref[0] + deg_ref[1] + 1.0)
    dis_ref[...] = dis
    g1_ref[...] = dis * jnp.dot(x, w1_ref[...], preferred_element_type=_F32)
    id1_ref[...] = jnp.dot(x, t1w_ref[...], preferred_element_type=_F32) + t1b_ref[...]


_tc_pre = pl.pallas_call(
    _pre_body,
    grid=(N // RB,),
    in_specs=[
        pl.BlockSpec((RB, D_IN), lambda i: (i, 0)),
        pl.BlockSpec((NC, RB, D_HID), lambda i: (0, i, 0)),
        pl.BlockSpec((D_IN, D_HID), lambda i: (0, 0)),
        pl.BlockSpec((D_IN, D_HID), lambda i: (0, 0)),
        pl.BlockSpec((1, D_HID), lambda i: (0, 0)),
    ],
    out_specs=[pl.BlockSpec((RB, D_HID), lambda i: (i, 0))] * 3,
    out_shape=[jax.ShapeDtypeStruct((N, D_HID), _F32)] * 3,
)


def _mid_body(s_ref, g1_ref, dis_ref, id1_ref, b1_ref, t2w_ref, t2b_ref,
              g2_ref, id2_ref):
    s = s_ref[0] + s_ref[1] + g1_ref[...]
    x1 = jnp.maximum(dis_ref[...] * s + b1_ref[...], 0.0) + id1_ref[...]
    g2_ref[...] = dis_ref[...] * x1
    id2_ref[...] = jnp.dot(x1, t2w_ref[...], preferred_element_type=_F32) + t2b_ref[...]


_tc_mid = pl.pallas_call(
    _mid_body,
    grid=(N // RB,),
    in_specs=[
        pl.BlockSpec((NC, RB, D_HID), lambda i: (0, i, 0)),
        pl.BlockSpec((RB, D_HID), lambda i: (i, 0)),
        pl.BlockSpec((RB, D_HID), lambda i: (i, 0)),
        pl.BlockSpec((RB, D_HID), lambda i: (i, 0)),
        pl.BlockSpec((1, D_HID), lambda i: (0, 0)),
        pl.BlockSpec((D_HID, D_OUT), lambda i: (0, 0)),
        pl.BlockSpec((1, D_OUT), lambda i: (0, 0)),
    ],
    out_specs=[
        pl.BlockSpec((RB, D_HID), lambda i: (i, 0)),
        pl.BlockSpec((RB, D_OUT), lambda i: (i, 0)),
    ],
    out_shape=[
        jax.ShapeDtypeStruct((N, D_HID), _F32),
        jax.ShapeDtypeStruct((N, D_OUT), _F32),
    ],
)


def _post_body(s_ref, g2_ref, dis_ref, w2_ref, b2_ref, id2_ref, out_ref):
    a2 = dis_ref[...] * (s_ref[0] + s_ref[1] + g2_ref[...])
    out_ref[...] = jnp.maximum(
        jnp.dot(a2, w2_ref[...], preferred_element_type=_F32) + b2_ref[...], 0.0
    ) + id2_ref[...]


_tc_post = pl.pallas_call(
    _post_body,
    grid=(N // RB,),
    in_specs=[
        pl.BlockSpec((NC, RB, D_HID), lambda i: (0, i, 0)),
        pl.BlockSpec((RB, D_HID), lambda i: (i, 0)),
        pl.BlockSpec((RB, D_HID), lambda i: (i, 0)),
        pl.BlockSpec((D_HID, D_OUT), lambda i: (0, 0)),
        pl.BlockSpec((1, D_OUT), lambda i: (0, 0)),
        pl.BlockSpec((RB, D_OUT), lambda i: (i, 0)),
    ],
    out_specs=pl.BlockSpec((RB, D_OUT), lambda i: (i, 0)),
    out_shape=jax.ShapeDtypeStruct((N, D_OUT), _F32),
)


def kernel(x, edge_index, W1, b1, W2, b2, T1w, T1b, T2w, T2b):
    src = edge_index[0].astype(jnp.int32)
    dst = edge_index[1].astype(jnp.int32)
    pad = EPAD - E
    src_p = jnp.concatenate([src, jnp.zeros((pad,), jnp.int32)]).reshape(NW, JCH, CHUNK)
    dst_p = jnp.concatenate([dst, jnp.full((pad,), N, jnp.int32)]).reshape(NW, JCH, CHUNK)

    ones = jnp.ones((CHUNK, D_HID), _F32)
    zeros = jnp.zeros((RPT, D_HID), _F32)
    b1r = b1.reshape(1, D_HID)
    t1br = T1b.reshape(1, D_HID)
    b2r = b2.reshape(1, D_OUT)
    t2br = T2b.reshape(1, D_OUT)

    degp = _sc_degree(dst_p, ones, zeros)
    dis2d, g1, id1 = _tc_pre(x, degp, W1, T1w, t1br)
    s1 = _sc_spmm(g1, src_p, dst_p, zeros)
    g2, id2 = _tc_mid(s1, g1, dis2d, id1, b1r, T2w, t2br)
    s2 = _sc_spmm(g2, src_p, dst_p, zeros)
    return _tc_post(s2, g2, dis2d, W2, b2r, id2)


# speedup vs baseline: 1.1955x; 1.1955x over previous
import functools

import jax
import jax.numpy as jnp
from jax import lax
from jax.experimental import pallas as pl
from jax.experimental.pallas import tpu as pltpu
from jax.experimental.pallas import tpu_sc as plsc

N = 10000
E = 320000
D_IN = 128
D_HID = 16
D_OUT = 128

NC = 2
NS = 16
NW = NC * NS
CHUNK = 128
JCH = 79
EW = JCH * CHUNK
EPAD = EW * NW
NPAD = 10240
RPT = NPAD // NS

_MESH = plsc.VectorSubcoreMesh(core_axis_name="c", subcore_axis_name="s")


_SC_PARAMS = pltpu.CompilerParams(use_tc_tiling_on_sc=False)


@functools.partial(
    pl.kernel,
    out_type=jax.ShapeDtypeStruct((NC, NPAD, D_HID), jnp.float32),
    mesh=_MESH,
    compiler_params=_SC_PARAMS,
    scratch_types=[
        pltpu.VMEM((JCH, CHUNK), jnp.int32),
        pltpu.VMEM((CHUNK, D_HID), jnp.float32),
        pltpu.VMEM_SHARED((NPAD, D_HID), jnp.float32),
        pltpu.SemaphoreType.DMA,
    ],
)
def _sc_degree(dst_hbm, ones_hbm, zeros_hbm, out_hbm, didx, ones_v, acc, sem):
    cid = lax.axis_index("c")
    sid = lax.axis_index("s")
    wid = sid * NC + cid
    pltpu.sync_copy(zeros_hbm, acc.at[pl.ds(sid * RPT, RPT)])
    pltpu.sync_copy(dst_hbm.at[wid], didx)
    pltpu.sync_copy(ones_hbm, ones_v)
    plsc.subcore_barrier()

    @pl.loop(0, JCH)
    def _(j):
        pltpu.async_copy(ones_v, acc.at[didx.at[j]], sem, add=True)

    @pl.loop(0, JCH)
    def _(j):
        pltpu.make_async_copy(ones_v, acc.at[didx.at[j]], sem).wait()

    plsc.subcore_barrier()
    pltpu.sync_copy(acc.at[pl.ds(sid * RPT, RPT)],
                    out_hbm.at[cid, pl.ds(sid * RPT, RPT)])


@functools.partial(
    pl.kernel,
    out_type=jax.ShapeDtypeStruct((NC, NPAD, D_HID), jnp.float32),
    mesh=_MESH,
    compiler_params=_SC_PARAMS,
    scratch_types=[
        pltpu.VMEM((JCH, CHUNK), jnp.int32),
        pltpu.VMEM((JCH, CHUNK), jnp.int32),
        pltpu.VMEM((4, CHUNK, D_HID), jnp.float32),
        pltpu.VMEM_SHARED((NPAD, D_HID), jnp.float32),
        [pltpu.SemaphoreType.DMA] * 4,
    ],
)
def _sc_spmm(g_hbm, src_hbm, dst_hbm, zeros_hbm, out_hbm,
             sidx, didx, rows, acc, gsem):
    cid = lax.axis_index("c")
    sid = lax.axis_index("s")
    wid = sid * NC + cid
    pltpu.sync_copy(zeros_hbm, acc.at[pl.ds(sid * RPT, RPT)])
    pltpu.sync_copy(src_hbm.at[wid], sidx)
    pltpu.sync_copy(dst_hbm.at[wid], didx)
    plsc.subcore_barrier()

    def _gather(j, b):
        pltpu.async_copy(g_hbm.at[sidx.at[j]], rows.at[b], gsem[b])

    def _wait_scatter(j, b):
        pltpu.make_async_copy(g_hbm.at[sidx.at[j]], rows.at[b], gsem[b]).wait()
        pltpu.sync_copy(rows.at[b], acc.at[didx.at[j]], add=True)

    _gather(0, 0)
    _gather(1, 1)
    _gather(2, 2)

    @pl.loop(0, JCH - 6, step=4)
    def _(j):
        _gather(j + 3, 3)
        _wait_scatter(j, 0)
        _gather(j + 4, 0)
        _wait_scatter(j + 1, 1)
        _gather(j + 5, 1)
        _wait_scatter(j + 2, 2)
        _gather(j + 6, 2)
        _wait_scatter(j + 3, 3)

    _wait_scatter(JCH - 3, 0)
    _wait_scatter(JCH - 2, 1)
    _wait_scatter(JCH - 1, 2)

    plsc.subcore_barrier()
    pltpu.sync_copy(acc.at[pl.ds(sid * RPT, RPT)],
                    out_hbm.at[cid, pl.ds(sid * RPT, RPT)])


RB = 2000
_F32 = jnp.float32


def _pre_body(x_ref, deg_ref, w1_ref, t1w_ref, t1b_ref,
              dis_ref, g1_ref, id1_ref):
    x = x_ref[...]
    dis = lax.rsqrt(deg_ref[0] + deg_ref[1] + 1.0)
    dis_ref[...] = dis
    g1_ref[...] = dis * jnp.dot(x, w1_ref[...], preferred_element_type=_F32)
    id1_ref[...] = jnp.dot(x, t1w_ref[...], preferred_element_type=_F32) + t1b_ref[...]


_tc_pre = pl.pallas_call(
    _pre_body,
    grid=(N // RB,),
    in_specs=[
        pl.BlockSpec((RB, D_IN), lambda i: (i, 0)),
        pl.BlockSpec((NC, RB, D_HID), lambda i: (0, i, 0)),
        pl.BlockSpec((D_IN, D_HID), lambda i: (0, 0)),
        pl.BlockSpec((D_IN, D_HID), lambda i: (0, 0)),
        pl.BlockSpec((1, D_HID), lambda i: (0, 0)),
    ],
    out_specs=[pl.BlockSpec((RB, D_HID), lambda i: (i, 0))] * 3,
    out_shape=[jax.ShapeDtypeStruct((N, D_HID), _F32)] * 3,
)


def _mid_body(s_ref, g1_ref, dis_ref, id1_ref, b1_ref, t2w_ref, t2b_ref,
              g2_ref, id2_ref):
    s = s_ref[0] + s_ref[1] + g1_ref[...]
    x1 = jnp.maximum(dis_ref[...] * s + b1_ref[...], 0.0) + id1_ref[...]
    g2_ref[...] = dis_ref[...] * x1
    id2_ref[...] = jnp.dot(x1, t2w_ref[...], preferred_element_type=_F32) + t2b_ref[...]


_tc_mid = pl.pallas_call(
    _mid_body,
    grid=(N // RB,),
    in_specs=[
        pl.BlockSpec((NC, RB, D_HID), lambda i: (0, i, 0)),
        pl.BlockSpec((RB, D_HID), lambda i: (i, 0)),
        pl.BlockSpec((RB, D_HID), lambda i: (i, 0)),
        pl.BlockSpec((RB, D_HID), lambda i: (i, 0)),
        pl.BlockSpec((1, D_HID), lambda i: (0, 0)),
        pl.BlockSpec((D_HID, D_OUT), lambda i: (0, 0)),
        pl.BlockSpec((1, D_OUT), lambda i: (0, 0)),
    ],
    out_specs=[
        pl.BlockSpec((RB, D_HID), lambda i: (i, 0)),
        pl.BlockSpec((RB, D_OUT), lambda i: (i, 0)),
    ],
    out_shape=[
        jax.ShapeDtypeStruct((N, D_HID), _F32),
        jax.ShapeDtypeStruct((N, D_OUT), _F32),
    ],
)


def _post_body(s_ref, g2_ref, dis_ref, w2_ref, b2_ref, id2_ref, out_ref):
    a2 = dis_ref[...] * (s_ref[0] + s_ref[1] + g2_ref[...])
    out_ref[...] = jnp.maximum(
        jnp.dot(a2, w2_ref[...], preferred_element_type=_F32) + b2_ref[...], 0.0
    ) + id2_ref[...]


_tc_post = pl.pallas_call(
    _post_body,
    grid=(N // RB,),
    in_specs=[
        pl.BlockSpec((NC, RB, D_HID), lambda i: (0, i, 0)),
        pl.BlockSpec((RB, D_HID), lambda i: (i, 0)),
        pl.BlockSpec((RB, D_HID), lambda i: (i, 0)),
        pl.BlockSpec((D_HID, D_OUT), lambda i: (0, 0)),
        pl.BlockSpec((1, D_OUT), lambda i: (0, 0)),
        pl.BlockSpec((RB, D_OUT), lambda i: (i, 0)),
    ],
    out_specs=pl.BlockSpec((RB, D_OUT), lambda i: (i, 0)),
    out_shape=jax.ShapeDtypeStruct((N, D_OUT), _F32),
)


def kernel(x, edge_index, W1, b1, W2, b2, T1w, T1b, T2w, T2b):
    src = edge_index[0].astype(jnp.int32)
    dst = edge_index[1].astype(jnp.int32)
    pad = EPAD - E
    src_p = jnp.concatenate([src, jnp.zeros((pad,), jnp.int32)]).reshape(NW, JCH, CHUNK)
    dst_p = jnp.concatenate([dst, jnp.full((pad,), N, jnp.int32)]).reshape(NW, JCH, CHUNK)

    ones = jnp.ones((CHUNK, D_HID), _F32)
    zeros = jnp.zeros((RPT, D_HID), _F32)
    b1r = b1.reshape(1, D_HID)
    t1br = T1b.reshape(1, D_HID)
    b2r = b2.reshape(1, D_OUT)
    t2br = T2b.reshape(1, D_OUT)

    degp = _sc_degree(dst_p, ones, zeros)
    dis2d, g1, id1 = _tc_pre(x, degp, W1, T1w, t1br)
    s1 = _sc_spmm(g1, src_p, dst_p, zeros)
    g2, id2 = _tc_mid(s1, g1, dis2d, id1, b1r, T2w, t2br)
    s2 = _sc_spmm(g2, src_p, dst_p, zeros)
    return _tc_post(s2, g2, dis2d, W2, b2r, id2)


# speedup vs baseline: 1.2221x; 1.0222x over previous
import functools

import jax
import jax.numpy as jnp
from jax import lax
from jax.experimental import pallas as pl
from jax.experimental.pallas import tpu as pltpu
from jax.experimental.pallas import tpu_sc as plsc

N = 10000
E = 320000
D_IN = 128
D_HID = 16
D_OUT = 128

NC = 2
NS = 16
NW = NC * NS
CHUNK = 128
JCH = 79
EW = JCH * CHUNK
EPAD = EW * NW
NPAD = 10240
RPT = NPAD // NS

_MESH = plsc.VectorSubcoreMesh(core_axis_name="c", subcore_axis_name="s")


_SC_PARAMS = pltpu.CompilerParams(use_tc_tiling_on_sc=False)


@functools.partial(
    pl.kernel,
    out_type=jax.ShapeDtypeStruct((NC, NPAD, D_HID), jnp.float32),
    mesh=_MESH,
    compiler_params=_SC_PARAMS,
    scratch_types=[
        pltpu.VMEM((JCH, CHUNK), jnp.int32),
        pltpu.VMEM((CHUNK, D_HID), jnp.float32),
        pltpu.VMEM_SHARED((NPAD, D_HID), jnp.float32),
        pltpu.SemaphoreType.DMA,
    ],
)
def _sc_degree(dst_hbm, ones_hbm, zeros_hbm, out_hbm, didx, ones_v, acc, sem):
    cid = lax.axis_index("c")
    sid = lax.axis_index("s")
    wid = sid * NC + cid
    pltpu.sync_copy(zeros_hbm, acc.at[pl.ds(sid * RPT, RPT)])
    pltpu.sync_copy(dst_hbm.at[wid], didx)
    pltpu.sync_copy(ones_hbm, ones_v)
    plsc.subcore_barrier()

    @pl.loop(0, JCH)
    def _(j):
        pltpu.async_copy(ones_v, acc.at[didx.at[j]], sem, add=True)

    @pl.loop(0, JCH)
    def _(j):
        pltpu.make_async_copy(ones_v, acc.at[didx.at[j]], sem).wait()

    plsc.subcore_barrier()
    pltpu.sync_copy(acc.at[pl.ds(sid * RPT, RPT)],
                    out_hbm.at[cid, pl.ds(sid * RPT, RPT)])


@functools.partial(
    pl.kernel,
    out_type=jax.ShapeDtypeStruct((NC, NPAD, D_HID), jnp.float32),
    mesh=_MESH,
    compiler_params=_SC_PARAMS,
    scratch_types=[
        pltpu.VMEM((JCH, CHUNK), jnp.int32),
        pltpu.VMEM((JCH, CHUNK), jnp.int32),
        pltpu.VMEM((5, CHUNK, D_HID), jnp.float32),
        pltpu.VMEM_SHARED((NPAD, D_HID), jnp.float32),
        [pltpu.SemaphoreType.DMA] * 5,
    ],
)
def _sc_spmm(g_hbm, src_hbm, dst_hbm, zeros_hbm, out_hbm,
             sidx, didx, rows, acc, gsem):
    cid = lax.axis_index("c")
    sid = lax.axis_index("s")
    wid = sid * NC + cid
    pltpu.sync_copy(zeros_hbm, acc.at[pl.ds(sid * RPT, RPT)])
    pltpu.sync_copy(src_hbm.at[wid], sidx)
    pltpu.sync_copy(dst_hbm.at[wid], didx)
    plsc.subcore_barrier()

    def _gather(j, b):
        pltpu.async_copy(g_hbm.at[sidx.at[j]], rows.at[b], gsem[b])

    def _wait_scatter(j, b):
        pltpu.make_async_copy(g_hbm.at[sidx.at[j]], rows.at[b], gsem[b]).wait()
        pltpu.sync_copy(rows.at[b], acc.at[didx.at[j]], add=True)

    _gather(0, 0)
    _gather(1, 1)
    _gather(2, 2)
    _gather(3, 3)

    @pl.loop(0, JCH - 8, step=5)
    def _(j):
        _gather(j + 4, 4)
        _wait_scatter(j, 0)
        _gather(j + 5, 0)
        _wait_scatter(j + 1, 1)
        _gather(j + 6, 1)
        _wait_scatter(j + 2, 2)
        _gather(j + 7, 2)
        _wait_scatter(j + 3, 3)
        _gather(j + 8, 3)
        _wait_scatter(j + 4, 4)

    _wait_scatter(JCH - 4, 0)
    _wait_scatter(JCH - 3, 1)
    _wait_scatter(JCH - 2, 2)
    _wait_scatter(JCH - 1, 3)

    plsc.subcore_barrier()
    pltpu.sync_copy(acc.at[pl.ds(sid * RPT, RPT)],
                    out_hbm.at[cid, pl.ds(sid * RPT, RPT)])


RB = 2000
_F32 = jnp.float32


def _pre_body(x_ref, deg_ref, w1_ref, t1w_ref, t1b_ref,
              dis_ref, g1_ref, id1_ref):
    x = x_ref[...]
    dis = lax.rsqrt(deg_ref[0] + deg_ref[1] + 1.0)
    dis_ref[...] = dis
    g1_ref[...] = dis * jnp.dot(x, w1_ref[...], preferred_element_type=_F32)
    id1_ref[...] = jnp.dot(x, t1w_ref[...], preferred_element_type=_F32) + t1b_ref[...]


_tc_pre = pl.pallas_call(
    _pre_body,
    grid=(N // RB,),
    in_specs=[
        pl.BlockSpec((RB, D_IN), lambda i: (i, 0)),
        pl.BlockSpec((NC, RB, D_HID), lambda i: (0, i, 0)),
        pl.BlockSpec((D_IN, D_HID), lambda i: (0, 0)),
        pl.BlockSpec((D_IN, D_HID), lambda i: (0, 0)),
        pl.BlockSpec((1, D_HID), lambda i: (0, 0)),
    ],
    out_specs=[pl.BlockSpec((RB, D_HID), lambda i: (i, 0))] * 3,
    out_shape=[jax.ShapeDtypeStruct((N, D_HID), _F32)] * 3,
)


def _mid_body(s_ref, g1_ref, dis_ref, id1_ref, b1_ref, t2w_ref, t2b_ref,
              g2_ref, id2_ref):
    s = s_ref[0] + s_ref[1] + g1_ref[...]
    x1 = jnp.maximum(dis_ref[...] * s + b1_ref[...], 0.0) + id1_ref[...]
    g2_ref[...] = dis_ref[...] * x1
    id2_ref[...] = jnp.dot(x1, t2w_ref[...], preferred_element_type=_F32) + t2b_ref[...]


_tc_mid = pl.pallas_call(
    _mid_body,
    grid=(N // RB,),
    in_specs=[
        pl.BlockSpec((NC, RB, D_HID), lambda i: (0, i, 0)),
        pl.BlockSpec((RB, D_HID), lambda i: (i, 0)),
        pl.BlockSpec((RB, D_HID), lambda i: (i, 0)),
        pl.BlockSpec((RB, D_HID), lambda i: (i, 0)),
        pl.BlockSpec((1, D_HID), lambda i: (0, 0)),
        pl.BlockSpec((D_HID, D_OUT), lambda i: (0, 0)),
        pl.BlockSpec((1, D_OUT), lambda i: (0, 0)),
    ],
    out_specs=[
        pl.BlockSpec((RB, D_HID), lambda i: (i, 0)),
        pl.BlockSpec((RB, D_OUT), lambda i: (i, 0)),
    ],
    out_shape=[
        jax.ShapeDtypeStruct((N, D_HID), _F32),
        jax.ShapeDtypeStruct((N, D_OUT), _F32),
    ],
)


def _post_body(s_ref, g2_ref, dis_ref, w2_ref, b2_ref, id2_ref, out_ref):
    a2 = dis_ref[...] * (s_ref[0] + s_ref[1] + g2_ref[...])
    out_ref[...] = jnp.maximum(
        jnp.dot(a2, w2_ref[...], preferred_element_type=_F32) + b2_ref[...], 0.0
    ) + id2_ref[...]


_tc_post = pl.pallas_call(
    _post_body,
    grid=(N // RB,),
    in_specs=[
        pl.BlockSpec((NC, RB, D_HID), lambda i: (0, i, 0)),
        pl.BlockSpec((RB, D_HID), lambda i: (i, 0)),
        pl.BlockSpec((RB, D_HID), lambda i: (i, 0)),
        pl.BlockSpec((D_HID, D_OUT), lambda i: (0, 0)),
        pl.BlockSpec((1, D_OUT), lambda i: (0, 0)),
        pl.BlockSpec((RB, D_OUT), lambda i: (i, 0)),
    ],
    out_specs=pl.BlockSpec((RB, D_OUT), lambda i: (i, 0)),
    out_shape=jax.ShapeDtypeStruct((N, D_OUT), _F32),
)


def kernel(x, edge_index, W1, b1, W2, b2, T1w, T1b, T2w, T2b):
    src = edge_index[0].astype(jnp.int32)
    dst = edge_index[1].astype(jnp.int32)
    pad = EPAD - E
    src_p = jnp.concatenate([src, jnp.zeros((pad,), jnp.int32)]).reshape(NW, JCH, CHUNK)
    dst_p = jnp.concatenate([dst, jnp.full((pad,), N, jnp.int32)]).reshape(NW, JCH, CHUNK)

    ones = jnp.ones((CHUNK, D_HID), _F32)
    zeros = jnp.zeros((RPT, D_HID), _F32)
    b1r = b1.reshape(1, D_HID)
    t1br = T1b.reshape(1, D_HID)
    b2r = b2.reshape(1, D_OUT)
    t2br = T2b.reshape(1, D_OUT)

    degp = _sc_degree(dst_p, ones, zeros)
    dis2d, g1, id1 = _tc_pre(x, degp, W1, T1w, t1br)
    s1 = _sc_spmm(g1, src_p, dst_p, zeros)
    g2, id2 = _tc_mid(s1, g1, dis2d, id1, b1r, T2w, t2br)
    s2 = _sc_spmm(g2, src_p, dst_p, zeros)
    return _tc_post(s2, g2, dis2d, W2, b2r, id2)


# speedup vs baseline: 1.2327x; 1.0087x over previous
import functools

import jax
import jax.numpy as jnp
from jax import lax
from jax.experimental import pallas as pl
from jax.experimental.pallas import tpu as pltpu
from jax.experimental.pallas import tpu_sc as plsc

N = 10000
E = 320000
D_IN = 128
D_HID = 16
D_OUT = 128

NC = 2
NS = 16
NW = NC * NS
CHUNK = 128
JCH = 79
EW = JCH * CHUNK
EPAD = EW * NW
NPAD = 10240
RPT = NPAD // NS

_MESH = plsc.VectorSubcoreMesh(core_axis_name="c", subcore_axis_name="s")


_SC_PARAMS = pltpu.CompilerParams(use_tc_tiling_on_sc=False)


@functools.partial(
    pl.kernel,
    out_type=jax.ShapeDtypeStruct((NC, NPAD, D_HID), jnp.float32),
    mesh=_MESH,
    compiler_params=_SC_PARAMS,
    scratch_types=[
        pltpu.VMEM((JCH, CHUNK), jnp.int32),
        pltpu.VMEM((CHUNK, D_HID), jnp.float32),
        pltpu.VMEM_SHARED((NPAD, D_HID), jnp.float32),
        pltpu.SemaphoreType.DMA,
    ],
)
def _sc_degree(dst_hbm, ones_hbm, zeros_hbm, out_hbm, didx, ones_v, acc, sem):
    cid = lax.axis_index("c")
    sid = lax.axis_index("s")
    wid = sid * NC + cid
    pltpu.sync_copy(zeros_hbm, acc.at[pl.ds(sid * RPT, RPT)])
    pltpu.sync_copy(dst_hbm.at[wid], didx)
    pltpu.sync_copy(ones_hbm, ones_v)
    plsc.subcore_barrier()

    @pl.loop(0, JCH)
    def _(j):
        pltpu.async_copy(ones_v, acc.at[didx.at[j]], sem, add=True)

    @pl.loop(0, JCH)
    def _(j):
        pltpu.make_async_copy(ones_v, acc.at[didx.at[j]], sem).wait()

    plsc.subcore_barrier()
    pltpu.sync_copy(acc.at[pl.ds(sid * RPT, RPT)],
                    out_hbm.at[cid, pl.ds(sid * RPT, RPT)])


@functools.partial(
    pl.kernel,
    out_type=jax.ShapeDtypeStruct((NC, NPAD, D_HID), jnp.float32),
    mesh=_MESH,
    compiler_params=_SC_PARAMS,
    scratch_types=[
        pltpu.VMEM((JCH, CHUNK), jnp.int32),
        pltpu.VMEM((JCH, CHUNK), jnp.int32),
        pltpu.VMEM((6, CHUNK, D_HID), jnp.float32),
        pltpu.VMEM_SHARED((NPAD, D_HID), jnp.float32),
        [pltpu.SemaphoreType.DMA] * 6,
    ],
)
def _sc_spmm(g_hbm, src_hbm, dst_hbm, zeros_hbm, out_hbm,
             sidx, didx, rows, acc, gsem):
    cid = lax.axis_index("c")
    sid = lax.axis_index("s")
    wid = sid * NC + cid
    pltpu.sync_copy(zeros_hbm, acc.at[pl.ds(sid * RPT, RPT)])
    pltpu.sync_copy(src_hbm.at[wid], sidx)
    pltpu.sync_copy(dst_hbm.at[wid], didx)
    plsc.subcore_barrier()

    def _gather(j, b):
        pltpu.async_copy(g_hbm.at[sidx.at[j]], rows.at[b], gsem[b])

    def _wait_scatter(j, b):
        pltpu.make_async_copy(g_hbm.at[sidx.at[j]], rows.at[b], gsem[b]).wait()
        pltpu.sync_copy(rows.at[b], acc.at[didx.at[j]], add=True)

    _gather(0, 0)
    _gather(1, 1)
    _gather(2, 2)
    _gather(3, 3)
    _gather(4, 4)

    @pl.loop(0, 67, step=6)
    def _(j):
        _gather(j + 5, 5)
        _wait_scatter(j, 0)
        _gather(j + 6, 0)
        _wait_scatter(j + 1, 1)
        _gather(j + 7, 1)
        _wait_scatter(j + 2, 2)
        _gather(j + 8, 2)
        _wait_scatter(j + 3, 3)
        _gather(j + 9, 3)
        _wait_scatter(j + 4, 4)
        _gather(j + 10, 4)
        _wait_scatter(j + 5, 5)

    _gather(77, 5)
    _wait_scatter(72, 0)
    _gather(78, 0)
    _wait_scatter(73, 1)
    _wait_scatter(74, 2)
    _wait_scatter(75, 3)
    _wait_scatter(76, 4)
    _wait_scatter(77, 5)
    _wait_scatter(78, 0)

    plsc.subcore_barrier()
    pltpu.sync_copy(acc.at[pl.ds(sid * RPT, RPT)],
                    out_hbm.at[cid, pl.ds(sid * RPT, RPT)])


RB = 2000
_F32 = jnp.float32


def _pre_body(x_ref, deg_ref, w1_ref, t1w_ref, t1b_ref,
              dis_ref, g1_ref, id1_ref):
    x = x_ref[...]
    dis = lax.rsqrt(deg_ref[0] + deg_ref[1] + 1.0)
    dis_ref[...] = dis
    g1_ref[...] = dis * jnp.dot(x, w1_ref[...], preferred_element_type=_F32)
    id1_ref[...] = jnp.dot(x, t1w_ref[...], preferred_element_type=_F32) + t1b_ref[...]


_tc_pre = pl.pallas_call(
    _pre_body,
    grid=(N // RB,),
    in_specs=[
        pl.BlockSpec((RB, D_IN), lambda i: (i, 0)),
        pl.BlockSpec((NC, RB, D_HID), lambda i: (0, i, 0)),
        pl.BlockSpec((D_IN, D_HID), lambda i: (0, 0)),
        pl.BlockSpec((D_IN, D_HID), lambda i: (0, 0)),
        pl.BlockSpec((1, D_HID), lambda i: (0, 0)),
    ],
    out_specs=[pl.BlockSpec((RB, D_HID), lambda i: (i, 0))] * 3,
    out_shape=[jax.ShapeDtypeStruct((N, D_HID), _F32)] * 3,
)


def _mid_body(s_ref, g1_ref, dis_ref, id1_ref, b1_ref, t2w_ref, t2b_ref,
              g2_ref, id2_ref):
    s = s_ref[0] + s_ref[1] + g1_ref[...]
    x1 = jnp.maximum(dis_ref[...] * s + b1_ref[...], 0.0) + id1_ref[...]
    g2_ref[...] = dis_ref[...] * x1
    id2_ref[...] = jnp.dot(x1, t2w_ref[...], preferred_element_type=_F32) + t2b_ref[...]


_tc_mid = pl.pallas_call(
    _mid_body,
    grid=(N // RB,),
    in_specs=[
        pl.BlockSpec((NC, RB, D_HID), lambda i: (0, i, 0)),
        pl.BlockSpec((RB, D_HID), lambda i: (i, 0)),
        pl.BlockSpec((RB, D_HID), lambda i: (i, 0)),
        pl.BlockSpec((RB, D_HID), lambda i: (i, 0)),
        pl.BlockSpec((1, D_HID), lambda i: (0, 0)),
        pl.BlockSpec((D_HID, D_OUT), lambda i: (0, 0)),
        pl.BlockSpec((1, D_OUT), lambda i: (0, 0)),
    ],
    out_specs=[
        pl.BlockSpec((RB, D_HID), lambda i: (i, 0)),
        pl.BlockSpec((RB, D_OUT), lambda i: (i, 0)),
    ],
    out_shape=[
        jax.ShapeDtypeStruct((N, D_HID), _F32),
        jax.ShapeDtypeStruct((N, D_OUT), _F32),
    ],
)


def _post_body(s_ref, g2_ref, dis_ref, w2_ref, b2_ref, id2_ref, out_ref):
    a2 = dis_ref[...] * (s_ref[0] + s_ref[1] + g2_ref[...])
    out_ref[...] = jnp.maximum(
        jnp.dot(a2, w2_ref[...], preferred_element_type=_F32) + b2_ref[...], 0.0
    ) + id2_ref[...]


_tc_post = pl.pallas_call(
    _post_body,
    grid=(N // RB,),
    in_specs=[
        pl.BlockSpec((NC, RB, D_HID), lambda i: (0, i, 0)),
        pl.BlockSpec((RB, D_HID), lambda i: (i, 0)),
        pl.BlockSpec((RB, D_HID), lambda i: (i, 0)),
        pl.BlockSpec((D_HID, D_OUT), lambda i: (0, 0)),
        pl.BlockSpec((1, D_OUT), lambda i: (0, 0)),
        pl.BlockSpec((RB, D_OUT), lambda i: (i, 0)),
    ],
    out_specs=pl.BlockSpec((RB, D_OUT), lambda i: (i, 0)),
    out_shape=jax.ShapeDtypeStruct((N, D_OUT), _F32),
)


def kernel(x, edge_index, W1, b1, W2, b2, T1w, T1b, T2w, T2b):
    src = edge_index[0].astype(jnp.int32)
    dst = edge_index[1].astype(jnp.int32)
    pad = EPAD - E
    src_p = jnp.concatenate([src, jnp.zeros((pad,), jnp.int32)]).reshape(NW, JCH, CHUNK)
    dst_p = jnp.concatenate([dst, jnp.full((pad,), N, jnp.int32)]).reshape(NW, JCH, CHUNK)

    ones = jnp.ones((CHUNK, D_HID), _F32)
    zeros = jnp.zeros((RPT, D_HID), _F32)
    b1r = b1.reshape(1, D_HID)
    t1br = T1b.reshape(1, D_HID)
    b2r = b2.reshape(1, D_OUT)
    t2br = T2b.reshape(1, D_OUT)

    degp = _sc_degree(dst_p, ones, zeros)
    dis2d, g1, id1 = _tc_pre(x, degp, W1, T1w, t1br)
    s1 = _sc_spmm(g1, src_p, dst_p, zeros)
    g2, id2 = _tc_mid(s1, g1, dis2d, id1, b1r, T2w, t2br)
    s2 = _sc_spmm(g2, src_p, dst_p, zeros)
    return _tc_post(s2, g2, dis2d, W2, b2r, id2)
